# baseline (speedup 1.0000x reference)
import functools

import jax
import jax.numpy as jnp
from jax import lax
from jax.experimental import pallas as pl
from jax.experimental.pallas import tpu as pltpu
from jax.experimental.pallas import tpu_sc as plsc

NUM_ITEM = 1000000
HIDDEN = 64
SEQ = 200
BATCH = 4096
PAD_IDX = 3

NC = 2
NS = 16
LANES = 16
NW = NC * NS
N = BATCH * SEQ
ROWS_PW = N // NW
CHUNK = 512
NCH = ROWS_PW // CHUNK
GSZ = 128
GPC = CHUNK // GSZ


def _emb_body(xf, table, posf, out_rows, mask_out, idx_all, pos_v, row_buf,
              mask_b, np_b, sem):
    wid = lax.axis_index("s") * NC + lax.axis_index("c")
    base = wid * ROWS_PW
    pltpu.sync_copy(xf.at[pl.ds(base, ROWS_PW)], idx_all)
    pltpu.sync_copy(posf, pos_v)

    def chunk_body(c, carry):
        cbase = base + c * CHUNK
        for j in range(GPC):
            pltpu.async_copy(
                table.at[idx_all.at[pl.ds(c * CHUNK + j * GSZ, GSZ)]],
                row_buf.at[pl.ds(j * GSZ, GSZ), :],
                sem,
            ).wait()
        for i in range(CHUNK // LANES):
            v = idx_all[pl.ds(c * CHUNK + i * LANES, LANES)]
            ispad = v == PAD_IDX
            mask_b[pl.ds(i * LANES, LANES)] = jnp.where(ispad, 1, 0)
            np_b[pl.ds(i * LANES, LANES)] = jnp.where(ispad, 0.0, 1.0)
        s0 = lax.rem(cbase, SEQ)

        def row_body(r, s):
            nps = np_b[pl.ds(r, LANES)][0]
            for j in range(HIDDEN // LANES):
                rv = row_buf[r, pl.ds(j * LANES, LANES)]
                pv = pos_v[pl.ds(s * HIDDEN + j * LANES, LANES)]
                row_buf[r, pl.ds(j * LANES, LANES)] = (rv + pv) * nps
            return jnp.where(s == SEQ - 1, 0, s + 1)

        lax.fori_loop(0, CHUNK, row_body, s0, unroll=False)
        pltpu.sync_copy(row_buf, out_rows.at[pl.ds(cbase, CHUNK), :])
        pltpu.sync_copy(mask_b, mask_out.at[pl.ds(cbase, CHUNK)])
        return carry

    lax.fori_loop(0, NCH, chunk_body, 0, unroll=False)


_emb_call = pl.kernel(
    _emb_body,
    out_type=[
        jax.ShapeDtypeStruct((N, HIDDEN), jnp.float32),
        jax.ShapeDtypeStruct((N,), jnp.int32),
    ],
    mesh=plsc.VectorSubcoreMesh(
        core_axis_name="c", subcore_axis_name="s", num_cores=NC,
        num_subcores=NS),
    scratch_types=[
        pltpu.VMEM((ROWS_PW,), jnp.int32),
        pltpu.VMEM((SEQ * HIDDEN,), jnp.float32),
        pltpu.VMEM((CHUNK, HIDDEN), jnp.float32),
        pltpu.VMEM((CHUNK,), jnp.int32),
        pltpu.VMEM((CHUNK + LANES,), jnp.float32),
        pltpu.SemaphoreType.DMA,
    ],
    compiler_params=pltpu.CompilerParams(use_tc_tiling_on_sc=False),
)


def kernel(x, item_table, pos_table):
    xf = x.reshape(N)
    posf = pos_table.reshape(SEQ * HIDDEN)
    out_rows, mask_i32 = _emb_call(xf, item_table, posf)
    input_emb = out_rows.reshape(BATCH, SEQ, HIDDEN)
    pad_masking = mask_i32.reshape(BATCH, SEQ).astype(bool)
    return (input_emb, pad_masking)


# speedup vs baseline: 1.1160x; 1.1160x over previous
import functools

import jax
import jax.numpy as jnp
from jax import lax
from jax.experimental import pallas as pl
from jax.experimental.pallas import tpu as pltpu
from jax.experimental.pallas import tpu_sc as plsc

NUM_ITEM = 1000000
HIDDEN = 64
SEQ = 200
BATCH = 4096
PAD_IDX = 3

NC = 2
NS = 16
LANES = 16
NW = NC * NS
N = BATCH * SEQ
ROWS_PW = N // NW
CHUNK = 512
NCH = ROWS_PW // CHUNK
GSZ = 128
GPC = CHUNK // GSZ
RUNROLL = 4


def _emb_body(xf, table, posf, out_rows, mask_out, idx_all, pos_v,
              row_bufs, mask_bs, np_b, gsems, osems):
    wid = lax.axis_index("s") * NC + lax.axis_index("c")
    base = wid * ROWS_PW
    pltpu.sync_copy(xf.at[pl.ds(base, ROWS_PW)], idx_all)
    pltpu.sync_copy(posf, pos_v)

    def start_gathers(c, buf, sem):
        for j in range(GPC):
            pltpu.async_copy(
                table.at[idx_all.at[pl.ds(c * CHUNK + j * GSZ, GSZ)]],
                buf.at[pl.ds(j * GSZ, GSZ), :],
                sem,
            )

    def drain_gathers(c, buf, sem):
        for j in range(GPC):
            pltpu.make_async_copy(
                table.at[idx_all.at[pl.ds(c * CHUNK + j * GSZ, GSZ)]],
                buf.at[pl.ds(j * GSZ, GSZ), :],
                sem,
            ).wait()

    def out_copy(c, buf, mbuf, sem):
        cbase = base + c * CHUNK
        return (
            pltpu.make_async_copy(buf, out_rows.at[pl.ds(cbase, CHUNK), :],
                                  sem),
            pltpu.make_async_copy(mbuf, mask_out.at[pl.ds(cbase, CHUNK)],
                                  sem),
        )

    start_gathers(0, row_bufs[0], gsems[0])

    def pair_body(cc, carry):
        for b in range(2):
            c = cc * 2 + b
            buf, mbuf, gsem, osem = row_bufs[b], mask_bs[b], gsems[b], osems[b]
            nb = 1 - b
            @pl.when(c >= 1)
            def _():
                for h in out_copy(c - 1, row_bufs[nb], mask_bs[nb],
                                  osems[nb]):
                    h.wait()
            @pl.when(c + 1 < NCH)
            def _():
                start_gathers(c + 1, row_bufs[nb], gsems[nb])
            for i in range(CHUNK // LANES):
                v = idx_all[pl.ds(c * CHUNK + i * LANES, LANES)]
                ispad = v == PAD_IDX
                mbuf[pl.ds(i * LANES, LANES)] = jnp.where(ispad, 1, 0)
                np_b[pl.ds(i * LANES, LANES)] = jnp.where(ispad, 0.0, 1.0)
            drain_gathers(c, buf, gsem)
            s0 = lax.rem(base + c * CHUNK, SEQ)

            def row_body(g, s):
                r0 = g * RUNROLL
                for u in range(RUNROLL):
                    r = r0 + u
                    su = s + u
                    su = jnp.where(su >= SEQ, su - SEQ, su)
                    nps = np_b[pl.ds(r, LANES)][0]
                    for j in range(HIDDEN // LANES):
                        rv = buf[r, pl.ds(j * LANES, LANES)]
                        pv = pos_v[pl.ds(su * HIDDEN + j * LANES, LANES)]
                        buf[r, pl.ds(j * LANES, LANES)] = (rv + pv) * nps
                nxt = s + RUNROLL
                return jnp.where(nxt >= SEQ, nxt - SEQ, nxt)

            lax.fori_loop(0, CHUNK // RUNROLL, row_body, s0, unroll=False)
            for h in out_copy(c, buf, mbuf, osem):
                h.start()
        return carry

    lax.fori_loop(0, NCH // 2, pair_body, 0, unroll=False)
    for h in out_copy(NCH - 1, row_bufs[1], mask_bs[1], osems[1]):
        h.wait()


_emb_call = pl.kernel(
    _emb_body,
    out_type=[
        jax.ShapeDtypeStruct((N, HIDDEN), jnp.float32),
        jax.ShapeDtypeStruct((N,), jnp.int32),
    ],
    mesh=plsc.VectorSubcoreMesh(
        core_axis_name="c", subcore_axis_name="s", num_cores=NC,
        num_subcores=NS),
    scratch_types=[
        pltpu.VMEM((ROWS_PW,), jnp.int32),
        pltpu.VMEM((SEQ * HIDDEN,), jnp.float32),
        [pltpu.VMEM((CHUNK, HIDDEN), jnp.float32)] * 2,
        [pltpu.VMEM((CHUNK,), jnp.int32)] * 2,
        pltpu.VMEM((CHUNK + LANES,), jnp.float32),
        [pltpu.SemaphoreType.DMA] * 2,
        [pltpu.SemaphoreType.DMA] * 2,
    ],
    compiler_params=pltpu.CompilerParams(use_tc_tiling_on_sc=False),
)


def kernel(x, item_table, pos_table):
    xf = x.reshape(N)
    posf = pos_table.reshape(SEQ * HIDDEN)
    out_rows, mask_i32 = _emb_call(xf, item_table, posf)
    input_emb = out_rows.reshape(BATCH, SEQ, HIDDEN)
    pad_masking = mask_i32.reshape(BATCH, SEQ).astype(bool)
    return (input_emb, pad_masking)
